# baseline (speedup 1.0000x reference)
import dataclasses
import functools

import jax
import jax.numpy as jnp
from jax import lax
from jax.experimental import pallas as pl
from jax.experimental.pallas import tpu as pltpu
from jax.experimental.pallas import tpu_sc as plsc

N = 50000
E = 800000
H = 32
NC = 10
B = 64

NUM_CORES = 2
NUM_SUBCORES = 16
NW = NUM_CORES * NUM_SUBCORES
E_PER_W = E // NW
GATHER_CHUNK = 1000
POS_CHUNK = 5000
SCAT_CHUNK = 4000
EB = 6400

_SC_PARAMS = pltpu.CompilerParams(use_tc_tiling_on_sc=False)
if "needs_layout_passes" in pltpu.CompilerParams.__dataclass_fields__:
  _SC_PARAMS = dataclasses.replace(_SC_PARAMS, needs_layout_passes=False)


def _vmesh():
  return plsc.VectorSubcoreMesh(core_axis_name="c", subcore_axis_name="s")


def _wid():
  return lax.axis_index("s") * NUM_CORES + lax.axis_index("c")


def _sc_gather2(ta, tb, src, dst):
  @functools.partial(
      pl.kernel,
      out_type=(
          jax.ShapeDtypeStruct((E, H), jnp.float32),
          jax.ShapeDtypeStruct((E, H), jnp.float32),
      ),
      mesh=_vmesh(),
      compiler_params=_SC_PARAMS,
      scratch_types=[
          pltpu.VMEM((GATHER_CHUNK,), jnp.int32),
          pltpu.VMEM((GATHER_CHUNK,), jnp.int32),
          pltpu.VMEM((GATHER_CHUNK, H), jnp.float32),
          pltpu.VMEM((GATHER_CHUNK, H), jnp.float32),
          pltpu.SemaphoreType.DMA,
          pltpu.SemaphoreType.DMA,
      ],
  )
  def gather_kernel(ta_hbm, tb_hbm, src_hbm, dst_hbm, ga_hbm, gb_hbm, si_v,
                    di_v, ra_v, rb_v, sem1, sem2):
    base = _wid() * E_PER_W

    @pl.loop(0, E_PER_W // GATHER_CHUNK)
    def _(j):
      off = base + j * GATHER_CHUNK
      pltpu.sync_copy(src_hbm.at[pl.ds(off, GATHER_CHUNK)], si_v)
      pltpu.sync_copy(dst_hbm.at[pl.ds(off, GATHER_CHUNK)], di_v)
      pltpu.async_copy(ta_hbm.at[si_v], ra_v, sem1).wait()
      pltpu.async_copy(tb_hbm.at[di_v], rb_v, sem2).wait()
      pltpu.sync_copy(ra_v, ga_hbm.at[pl.ds(off, GATHER_CHUNK)])
      pltpu.sync_copy(rb_v, gb_hbm.at[pl.ds(off, GATHER_CHUNK)])

  return gather_kernel(ta, tb, src, dst)


def _pair_rmw(acc0_v, acc1_v, idx_v, val_v, g):
  idx0 = idx_v[pl.ds(g * 32, 16)]
  vals0 = val_v[pl.ds(g * 32, 16)]
  idx1 = idx_v[pl.ds(g * 32 + 16, 16)]
  vals1 = val_v[pl.ds(g * 32 + 16, 16)]
  c0 = plsc.load_gather(acc0_v, [idx0])
  c1 = plsc.load_gather(acc1_v, [idx1])
  plsc.store_scatter(acc0_v, [idx0], jnp.maximum(c0, vals0))
  plsc.store_scatter(acc1_v, [idx1], jnp.maximum(c1, vals1))
  g0 = plsc.load_gather(acc0_v, [idx0])
  g1 = plsc.load_gather(acc1_v, [idx1])
  p0 = jnp.where(g0 < vals0, 1, 0)
  p1 = jnp.where(g1 < vals1, 1, 0)

  def cond(st):
    return (lax.reduce_max(st[0], (0,)) | lax.reduce_max(st[1], (0,))) > 0

  def body(st):
    m0 = st[0] > 0
    r0 = plsc.load_gather(acc0_v, [idx0])
    plsc.store_scatter(acc0_v, [idx0], jnp.maximum(r0, vals0), mask=m0)
    m1 = st[1] > 0
    r1 = plsc.load_gather(acc1_v, [idx1])
    plsc.store_scatter(acc1_v, [idx1], jnp.maximum(r1, vals1), mask=m1)
    q0 = plsc.load_gather(acc0_v, [idx0])
    q1 = plsc.load_gather(acc1_v, [idx1])
    return (jnp.where(m0 & (q0 < vals0), 1, 0),
            jnp.where(m1 & (q1 < vals1), 1, 0))

  lax.while_loop(cond, body, (p0, p1))


def _scatter_stream(mT_hbm, dst_hbm, t, bufs):
  nch = E // SCAT_CHUNK

  for b in range(2):
    acc, idx_v, val_v, sem_i, sem_v = bufs[b]
    off = b * SCAT_CHUNK
    pltpu.async_copy(dst_hbm.at[pl.ds(off, SCAT_CHUNK)], idx_v, sem_i)
    pltpu.async_copy(mT_hbm.at[t, pl.ds(off, SCAT_CHUNK)], val_v, sem_v)

  @pl.loop(0, nch, step=2)
  def _(c0):
    for b in range(2):
      acc, idx_v, val_v, sem_i, sem_v = bufs[b]
      c = c0 + b
      off = c * SCAT_CHUNK
      pltpu.make_async_copy(dst_hbm.at[pl.ds(off, SCAT_CHUNK)], idx_v,
                            sem_i).wait()
      pltpu.make_async_copy(mT_hbm.at[t, pl.ds(off, SCAT_CHUNK)], val_v,
                            sem_v).wait()

      @pl.loop(0, SCAT_CHUNK // 32)
      def _(g):
        _pair_rmw(bufs[b][0], bufs[1 - b][0], idx_v, val_v, g)

      @pl.when(c + 2 < nch)
      def _():
        off2 = off + 2 * SCAT_CHUNK
        pltpu.async_copy(dst_hbm.at[pl.ds(off2, SCAT_CHUNK)], idx_v, sem_i)
        pltpu.async_copy(mT_hbm.at[t, pl.ds(off2, SCAT_CHUNK)], val_v, sem_v)


def _sc_scatter_max(mT, dst):
  @functools.partial(
      pl.kernel,
      out_type=jax.ShapeDtypeStruct((H, N), jnp.float32),
      mesh=_vmesh(),
      compiler_params=_SC_PARAMS,
      scratch_types=[
          pltpu.VMEM((N,), jnp.float32),
          pltpu.VMEM((N,), jnp.float32),
          pltpu.VMEM((SCAT_CHUNK,), jnp.int32),
          pltpu.VMEM((SCAT_CHUNK,), jnp.int32),
          pltpu.VMEM((SCAT_CHUNK,), jnp.float32),
          pltpu.VMEM((SCAT_CHUNK,), jnp.float32),
          pltpu.SemaphoreType.DMA,
          pltpu.SemaphoreType.DMA,
          pltpu.SemaphoreType.DMA,
          pltpu.SemaphoreType.DMA,
      ],
  )
  def scatter_kernel(mT_hbm, dst_hbm, out_hbm, acc0_v, acc1_v, idx0_v, idx1_v,
                     val0_v, val1_v, si0, si1, sv0, sv1):
    t = _wid()
    zeros16 = jnp.zeros((16,), jnp.float32)

    @pl.loop(0, N // 16)
    def _(i):
      acc0_v[pl.ds(i * 16, 16)] = zeros16
      acc1_v[pl.ds(i * 16, 16)] = zeros16

    bufs = ((acc0_v, idx0_v, val0_v, si0, sv0),
            (acc1_v, idx1_v, val1_v, si1, sv1))
    _scatter_stream(mT_hbm, dst_hbm, t, bufs)

    @pl.loop(0, N // 16)
    def _(i):
      s = pl.ds(i * 16, 16)
      acc0_v[s] = jnp.maximum(acc0_v[s], acc1_v[s])

    pltpu.sync_copy(acc0_v, out_hbm.at[t])

  return scatter_kernel(mT, dst)


def _sc_scatter_max_pool(mT, dst, starts):
  @functools.partial(
      pl.kernel,
      out_type=jax.ShapeDtypeStruct((H, B), jnp.float32),
      mesh=_vmesh(),
      compiler_params=_SC_PARAMS,
      scratch_types=[
          pltpu.VMEM((N + 16,), jnp.float32),
          pltpu.VMEM((N + 16,), jnp.float32),
          pltpu.VMEM((SCAT_CHUNK,), jnp.int32),
          pltpu.VMEM((SCAT_CHUNK,), jnp.int32),
          pltpu.VMEM((SCAT_CHUNK,), jnp.float32),
          pltpu.VMEM((SCAT_CHUNK,), jnp.float32),
          pltpu.VMEM((B,), jnp.float32),
          pltpu.VMEM((80,), jnp.int32),
          pltpu.SemaphoreType.DMA,
          pltpu.SemaphoreType.DMA,
          pltpu.SemaphoreType.DMA,
          pltpu.SemaphoreType.DMA,
      ],
  )
  def scatter_kernel(mT_hbm, dst_hbm, starts_hbm, out_hbm, acc_v, acc1_v,
                     idx0_v, idx1_v, val0_v, val1_v, g_v, starts_v, si0, si1,
                     sv0, sv1):
    t = _wid()
    zeros16 = jnp.zeros((16,), jnp.float32)
    pltpu.sync_copy(starts_hbm, starts_v)

    @pl.loop(0, (N + 16) // 16)
    def _(i):
      acc_v[pl.ds(i * 16, 16)] = zeros16
      acc1_v[pl.ds(i * 16, 16)] = zeros16

    bufs = ((acc_v, idx0_v, val0_v, si0, sv0),
            (acc1_v, idx1_v, val1_v, si1, sv1))
    _scatter_stream(mT_hbm, dst_hbm, t, bufs)

    @pl.loop(0, N // 16)
    def _(i):
      s = pl.ds(i * 16, 16)
      acc_v[s] = jnp.maximum(acc_v[s], acc1_v[s])

    lanes = lax.iota(jnp.int32, 16)

    for grp in range(B // 16):
      gv = zeros16
      for b2 in range(16):
        b = grp * 16 + b2
        s = starts_v[pl.ds((b // 16) * 16, 16)][b % 16]
        e = starts_v[pl.ds(((b + 1) // 16) * 16, 16)][(b + 1) % 16]
        n = e - s
        nfull = n // 16

        def seg_body(j, m, s=s):
          return jnp.maximum(m, acc_v[pl.ds(s + j * 16, 16)])

        m = lax.fori_loop(0, nfull, seg_body, zeros16)
        rem = n - nfull * 16
        v = acc_v[pl.ds(s + nfull * 16, 16)]
        m = jnp.maximum(m, jnp.where(lanes < rem, v, 0.0))
        gv = jnp.where(lanes == b2, lax.reduce_max(m, (0,)), gv)
      g_v[pl.ds(grp * 16, 16)] = gv

    pltpu.sync_copy(g_v, out_hbm.at[t])

  return scatter_kernel(mT, dst, starts)


def _tc_nodeproj(p8, w1a, w1b, w2b):
  def kernel(p_ref, wa_ref, wb_ref, wc_ref, a_ref, b_ref, c_ref):
    p = p_ref[...]
    a_ref[...] = jnp.dot(p, wa_ref[...], preferred_element_type=jnp.float32)
    b_ref[...] = jnp.dot(p, wb_ref[...], preferred_element_type=jnp.float32)
    c_ref[...] = jnp.dot(p, wc_ref[...], preferred_element_type=jnp.float32)

  nb = 10000
  return pl.pallas_call(
      kernel,
      grid=(N // nb,),
      in_specs=[
          pl.BlockSpec((nb, 8), lambda i: (i, 0)),
          pl.BlockSpec((8, H), lambda i: (0, 0)),
          pl.BlockSpec((8, H), lambda i: (0, 0)),
          pl.BlockSpec((8, H), lambda i: (0, 0)),
      ],
      out_specs=[
          pl.BlockSpec((nb, H), lambda i: (i, 0)),
          pl.BlockSpec((nb, H), lambda i: (i, 0)),
          pl.BlockSpec((nb, H), lambda i: (i, 0)),
      ],
      out_shape=[
          jax.ShapeDtypeStruct((N, H), jnp.float32),
          jax.ShapeDtypeStruct((N, H), jnp.float32),
          jax.ShapeDtypeStruct((N, H), jnp.float32),
      ],
  )(p8, w1a, w1b, w2b)


def _tc_a2(h1T, p2, w2a):
  def kernel(ht_ref, p2_ref, w_ref, out_ref):
    z = lax.dot_general(ht_ref[...], w_ref[...], (((0,), (0,)), ((), ())),
                        preferred_element_type=jnp.float32)
    out_ref[...] = z + p2_ref[...]

  return pl.pallas_call(
      kernel,
      in_specs=[
          pl.BlockSpec((H, N), lambda: (0, 0)),
          pl.BlockSpec((N, H), lambda: (0, 0)),
          pl.BlockSpec((H, H), lambda: (0, 0)),
      ],
      out_specs=pl.BlockSpec((N, H), lambda: (0, 0)),
      out_shape=jax.ShapeDtypeStruct((N, H), jnp.float32),
  )(h1T, p2, w2a)


def _tc_stats(ga, gb, bias, sub):
  def kernel(a_ref, b_ref, bias_ref, out_ref):
    i = pl.program_id(0)
    if sub:
      x = a_ref[...] - b_ref[...] + bias_ref[0]
    else:
      x = a_ref[...] + b_ref[...] + bias_ref[0]

    @pl.when(i == 0)
    def _():
      out_ref[...] = jnp.zeros_like(out_ref)

    s0 = jnp.sum(x, axis=0)[None]
    s1 = jnp.sum(x * x, axis=0)[None]
    out_ref[...] += jnp.concatenate(
        [s0, s1, jnp.zeros((6, H), jnp.float32)], axis=0)

  return pl.pallas_call(
      kernel,
      grid=(E // EB,),
      in_specs=[
          pl.BlockSpec((EB, H), lambda i: (i, 0)),
          pl.BlockSpec((EB, H), lambda i: (i, 0)),
          pl.BlockSpec((1, H), lambda i: (0, 0)),
      ],
      out_specs=pl.BlockSpec((8, H), lambda i: (0, 0)),
      out_shape=jax.ShapeDtypeStruct((8, H), jnp.float32),
  )(ga, gb, bias)


def _tc_messages(ga, gb, bias, sub, sums, gamma, beta, w2, b2):
  def kernel(a_ref, b_ref, bias_ref, sums_ref, g_ref, bt_ref, w2_ref, b2_ref,
             out_ref):
    if sub:
      x = a_ref[...] - b_ref[...] + bias_ref[0]
    else:
      x = a_ref[...] + b_ref[...] + bias_ref[0]
    mu = sums_ref[0] * (1.0 / E)
    var = sums_ref[1] * (1.0 / E) - mu * mu
    scale = g_ref[0] * lax.rsqrt(var + 1e-5)
    shift = bt_ref[0] - mu * scale
    xh = jnp.maximum(x * scale + shift, 0.0)
    yT = lax.dot_general(w2_ref[...], xh, (((0,), (1,)), ((), ())),
                         preferred_element_type=jnp.float32)
    out_ref[...] = yT + b2_ref[...].T

  return pl.pallas_call(
      kernel,
      grid=(E // EB,),
      in_specs=[
          pl.BlockSpec((EB, H), lambda i: (i, 0)),
          pl.BlockSpec((EB, H), lambda i: (i, 0)),
          pl.BlockSpec((1, H), lambda i: (0, 0)),
          pl.BlockSpec((8, H), lambda i: (0, 0)),
          pl.BlockSpec((1, H), lambda i: (0, 0)),
          pl.BlockSpec((1, H), lambda i: (0, 0)),
          pl.BlockSpec((H, H), lambda i: (0, 0)),
          pl.BlockSpec((1, H), lambda i: (0, 0)),
      ],
      out_specs=pl.BlockSpec((H, EB), lambda i: (0, i)),
      out_shape=jax.ShapeDtypeStruct((H, E), jnp.float32),
  )(ga, gb, bias, sums, gamma, beta, w2, b2)


def _tc_classifier(gT, cls_w, cls_b):
  def kernel(g_ref, w_ref, b_ref, out_ref):
    g = g_ref[...].T
    out_ref[...] = (
        jnp.dot(g, w_ref[...], preferred_element_type=jnp.float32) + b_ref[0])

  return pl.pallas_call(
      kernel,
      in_specs=[
          pl.BlockSpec((H, B), lambda: (0, 0)),
          pl.BlockSpec((H, NC), lambda: (0, 0)),
          pl.BlockSpec((1, NC), lambda: (0, 0)),
      ],
      out_specs=pl.BlockSpec((B, NC), lambda: (0, 0)),
      out_shape=jax.ShapeDtypeStruct((B, NC), jnp.float32),
  )(gT, cls_w, cls_b)


def kernel(pos, edge_index, batch, c1_w1, c1_b1, c1_gamma, c1_beta, c1_w2,
           c1_b2, c2_w1, c2_b1, c2_gamma, c2_beta, c2_w2, c2_b2, cls_w, cls_b):
  src = edge_index[0]
  dst = edge_index[1]

  z5 = jnp.zeros((5, H), jnp.float32)
  w1a = jnp.concatenate([c1_w1[0:3] + c1_w1[3:6], z5], axis=0)
  w1b = jnp.concatenate([-c1_w1[3:6], z5], axis=0)
  w2b = jnp.concatenate([c2_w1[H:H + 3], z5], axis=0)
  w2a = c2_w1[0:H]
  p8 = jnp.pad(pos, ((0, 0), (0, 5)))
  starts = jnp.searchsorted(batch, jnp.arange(B + 1, dtype=jnp.int32),
                            side="left").astype(jnp.int32)
  starts = jnp.pad(starts, (0, 15), constant_values=N)

  a1, b1t, p2 = _tc_nodeproj(p8, w1a, w1b, w2b)

  ga1, gb1 = _sc_gather2(a1, b1t, src, dst)
  sums1 = _tc_stats(ga1, gb1, c1_b1[None], sub=False)
  m1T = _tc_messages(ga1, gb1, c1_b1[None], False, sums1, c1_gamma[None],
                     c1_beta[None], c1_w2, c1_b2[None])
  h1T = _sc_scatter_max(m1T, dst)

  a2 = _tc_a2(h1T, p2, w2a)
  ga2, gb2 = _sc_gather2(a2, p2, src, dst)
  sums2 = _tc_stats(ga2, gb2, c2_b1[None], sub=True)
  m2T = _tc_messages(ga2, gb2, c2_b1[None], True, sums2, c2_gamma[None],
                     c2_beta[None], c2_w2, c2_b2[None])
  gT = _sc_scatter_max_pool(m2T, dst, starts)

  return _tc_classifier(gT, cls_w, cls_b[None])
